# baseline (speedup 1.0000x reference)
import functools

import jax
import jax.numpy as jnp
from jax import lax
from jax.experimental import pallas as pl
from jax.experimental.pallas import tpu as pltpu
from jax.experimental.pallas import tpu_sc as plsc

N = 10000
H = 128
E = 320000
G = 64
L = 3

NC = 2
NS = 16
NTILES = NC * NS
CHUNK = 64
NCHUNK = 162
EPAD = NTILES * NCHUNK * CHUNK
NPAD = 10240
ZROWS = NPAD // NS
RCHUNK = 128


def _sc_aggregate_body(rh_hbm, srcv_hbm, dstv_hbm, out_hbm,
                       srcv, dstv, rows, aggr, sems):
    c = lax.axis_index("c")
    s = lax.axis_index("s")

    pltpu.sync_copy(srcv_hbm.at[c, s], srcv)
    pltpu.sync_copy(dstv_hbm.at[c, s], dstv)

    z16 = jnp.zeros((16,), jnp.float32)

    def _zero_row(i, carry):
        for k in range(H // 16):
            rows[0, i, pl.ds(k * 16, 16)] = z16
        return carry

    lax.fori_loop(0, CHUNK, _zero_row, 0)

    def _zero_slab(k, carry):
        pltpu.sync_copy(rows.at[0], aggr.at[pl.ds(s * ZROWS + k * CHUNK, CHUNK)])
        return carry

    lax.fori_loop(0, ZROWS // CHUNK, _zero_slab, 0)
    plsc.subcore_barrier()

    def _gather(row, half, b):
        pltpu.async_copy(
            rh_hbm.at[srcv.at[row, pl.ds(half * CHUNK, CHUNK)]],
            rows.at[b], sems.at[b])

    def _wait(b):
        pltpu.make_async_copy(
            rh_hbm.at[srcv.at[0, pl.ds(0, CHUNK)]],
            rows.at[b], sems.at[b]).wait()

    for slot in range(3):
        _gather(slot // 2, slot % 2, slot)

    def _round(g, carry):
        for slot in range(6):
            b = slot % 3
            j = 6 * g + slot
            nxt = slot + 3
            _wait(b)
            pltpu.sync_copy(rows.at[b],
                            aggr.at[dstv.at[3 * g + slot // 2,
                                            pl.ds((slot % 2) * CHUNK, CHUNK)]],
                            add=True)

            @pl.when(j + 3 < NCHUNK)
            def _():
                _gather(3 * g + nxt // 2, nxt % 2, b)

        return carry

    lax.fori_loop(0, NCHUNK // 6, _round, 0)
    plsc.subcore_barrier()

    for k in range(ZROWS // RCHUNK):
        r0 = s * ZROWS + k * RCHUNK
        pltpu.sync_copy(aggr.at[pl.ds(r0, RCHUNK)], out_hbm.at[c, pl.ds(r0, RCHUNK)])


@functools.cache
def _sc_aggregate():
    return pl.kernel(
        _sc_aggregate_body,
        out_type=jax.ShapeDtypeStruct((NC, NPAD, H), jnp.float32),
        mesh=plsc.VectorSubcoreMesh(core_axis_name="c", subcore_axis_name="s",
                                    num_cores=NC, num_subcores=NS),
        scratch_types=[
            pltpu.VMEM((NCHUNK // 2, 2 * CHUNK), jnp.int32),
            pltpu.VMEM((NCHUNK // 2, 2 * CHUNK), jnp.int32),
            pltpu.VMEM((3, CHUNK, H), jnp.float32),
            pltpu.VMEM_SHARED((NPAD, H), jnp.float32),
            pltpu.SemaphoreType.DMA((3,)),
        ],
    )


BLK = 2000


def _inproj_body(x_ref, w_ref, b_ref, h_ref, rh_ref):
    z = jnp.dot(x_ref[...], w_ref[...], preferred_element_type=jnp.float32)
    z = z + b_ref[...]
    hz = z * jax.nn.sigmoid(z)
    h_ref[...] = hz
    rh_ref[...] = jnp.maximum(hz, 0.0)


_inproj = pl.pallas_call(
    _inproj_body,
    grid=(N // BLK,),
    in_specs=[
        pl.BlockSpec((BLK, H), lambda i: (i, 0)),
        pl.BlockSpec((H, H), lambda i: (0, 0)),
        pl.BlockSpec((1, H), lambda i: (0, 0)),
    ],
    out_specs=[pl.BlockSpec((BLK, H), lambda i: (i, 0))] * 2,
    out_shape=[jax.ShapeDtypeStruct((N, H), jnp.float32)] * 2,
)


def _layer_body(h_ref, p_ref, w_ref, b_ref, ga_ref, be_ref, out_ref, rh_ref):
    h0 = h_ref[...]
    p = p_ref[...]
    t = h0 + p[0] + p[1]
    z = jnp.dot(t, w_ref[...], preferred_element_type=jnp.float32)
    z = z + b_ref[...]
    mu = jnp.mean(z, axis=-1, keepdims=True)
    zc = z - mu
    var = jnp.mean(zc * zc, axis=-1, keepdims=True)
    zn = zc * lax.rsqrt(var + 1e-5) * ga_ref[...] + be_ref[...]
    zs = zn * jax.nn.sigmoid(zn)
    hn = zs + h0
    out_ref[...] = hn
    rh_ref[...] = jnp.maximum(hn, 0.0)


_layer = pl.pallas_call(
    _layer_body,
    grid=(N // BLK,),
    in_specs=[
        pl.BlockSpec((BLK, H), lambda i: (i, 0)),
        pl.BlockSpec((NC, BLK, H), lambda i: (0, i, 0)),
        pl.BlockSpec((H, H), lambda i: (0, 0)),
        pl.BlockSpec((1, H), lambda i: (0, 0)),
        pl.BlockSpec((1, H), lambda i: (0, 0)),
        pl.BlockSpec((1, H), lambda i: (0, 0)),
    ],
    out_specs=[pl.BlockSpec((BLK, H), lambda i: (i, 0))] * 2,
    out_shape=[jax.ShapeDtypeStruct((N, H), jnp.float32)] * 2,
)


def _pool_body(h_ref, batch_ref, out_ref):
    h = h_ref[...]
    bid = batch_ref[...]
    gids = lax.broadcasted_iota(jnp.int32, (N, G), 1)
    m = (bid == gids).astype(jnp.float32)
    sums = lax.dot_general(m, h, (((0,), (0,)), ((), ())),
                           preferred_element_type=jnp.float32)
    ones = jnp.ones((N, 1), jnp.float32)
    counts = lax.dot_general(m, ones, (((0,), (0,)), ((), ())),
                             preferred_element_type=jnp.float32)
    out_ref[...] = sums / jnp.maximum(counts, 1.0)


_pool = pl.pallas_call(
    _pool_body,
    out_shape=jax.ShapeDtypeStruct((G, H), jnp.float32),
)


def kernel(x, edge_index, batch, W_in, b_in, W, b, gamma, beta):
    src = edge_index[0]
    dst = edge_index[1]
    pad = EPAD - E
    src_p = jnp.concatenate([src, jnp.zeros((pad,), jnp.int32)])
    trash = N + jnp.arange(pad, dtype=jnp.int32) % (NPAD - N)
    dst_p = jnp.concatenate([dst, trash])
    srcv = src_p.reshape(NC, NS, NCHUNK // 2, 2 * CHUNK)
    dstv = dst_p.reshape(NC, NS, NCHUNK // 2, 2 * CHUNK)

    h, rh = _inproj(x, W_in, b_in.reshape(1, H))
    for l in range(L):
        parts = _sc_aggregate()(rh, srcv, dstv)
        h, rh = _layer(h, parts, W[l], b[l].reshape(1, H),
                       gamma[l].reshape(1, H), beta[l].reshape(1, H))
    return _pool(h, batch.reshape(N, 1))


# speedup vs baseline: 1.0142x; 1.0142x over previous
import functools

import jax
import jax.numpy as jnp
from jax import lax
from jax.experimental import pallas as pl
from jax.experimental.pallas import tpu as pltpu
from jax.experimental.pallas import tpu_sc as plsc

N = 10000
H = 128
E = 320000
G = 64
L = 3

NC = 2
NS = 16
NTILES = NC * NS
CHUNK = 64
NCHUNK = 162
EPAD = NTILES * NCHUNK * CHUNK
NPAD = 10240
ZROWS = NPAD // NS
RCHUNK = 128


def _sc_aggregate_body(rh_hbm, srcv_hbm, dstv_hbm, out_hbm,
                       srcv, dstv, rows, aggr, sems):
    c = lax.axis_index("c")
    s = lax.axis_index("s")

    with jax.named_scope("sc_load_slabs"):
        pltpu.sync_copy(srcv_hbm.at[c, s], srcv)
        pltpu.sync_copy(dstv_hbm.at[c, s], dstv)

    z16 = jnp.zeros((16,), jnp.float32)

    def _zero_row(i, carry):
        for k in range(H // 16):
            rows[0, i, pl.ds(k * 16, 16)] = z16
        return carry

    with jax.named_scope("sc_zero"):
        lax.fori_loop(0, CHUNK, _zero_row, 0)

        def _zero_slab(k, carry):
            pltpu.sync_copy(rows.at[0],
                            aggr.at[pl.ds(s * ZROWS + k * CHUNK, CHUNK)])
            return carry

        lax.fori_loop(0, ZROWS // CHUNK, _zero_slab, 0)
        plsc.subcore_barrier()

    def _gather(row, half, b):
        pltpu.async_copy(
            rh_hbm.at[srcv.at[row, pl.ds(half * CHUNK, CHUNK)]],
            rows.at[b], sems.at[b])

    def _wait(b):
        pltpu.make_async_copy(
            rh_hbm.at[srcv.at[0, pl.ds(0, CHUNK)]],
            rows.at[b], sems.at[b]).wait()

    def _scatter(row, half, b):
        pltpu.sync_copy(rows.at[b],
                        aggr.at[dstv.at[row, pl.ds(half * CHUNK, CHUNK)]],
                        add=True)

    with jax.named_scope("sc_mainloop"):
        _gather(0, 0, 0)
        _gather(0, 1, 1)

        def _round(g, carry):
            _wait(0)
            _scatter(g, 0, 0)

            @pl.when(g < NCHUNK // 2 - 1)
            def _():
                _gather(g + 1, 0, 0)

            _wait(1)
            _scatter(g, 1, 1)

            @pl.when(g < NCHUNK // 2 - 1)
            def _():
                _gather(g + 1, 1, 1)

            return carry

        lax.fori_loop(0, NCHUNK // 2, _round, 0)
        plsc.subcore_barrier()

    with jax.named_scope("sc_writeback"):
        for k in range(ZROWS // RCHUNK):
            r0 = s * ZROWS + k * RCHUNK
            pltpu.sync_copy(aggr.at[pl.ds(r0, RCHUNK)],
                            out_hbm.at[c, pl.ds(r0, RCHUNK)])


@functools.cache
def _sc_aggregate():
    return pl.kernel(
        _sc_aggregate_body,
        out_type=jax.ShapeDtypeStruct((NC, NPAD, H), jnp.float32),
        mesh=plsc.VectorSubcoreMesh(core_axis_name="c", subcore_axis_name="s",
                                    num_cores=NC, num_subcores=NS),
        scratch_types=[
            pltpu.VMEM((NCHUNK // 2, 2 * CHUNK), jnp.int32),
            pltpu.VMEM((NCHUNK // 2, 2 * CHUNK), jnp.int32),
            pltpu.VMEM((2, CHUNK, H), jnp.float32),
            pltpu.VMEM_SHARED((NPAD, H), jnp.float32),
            pltpu.SemaphoreType.DMA((2,)),
        ],
    )


BLK = 2000


def _inproj_body(x_ref, w_ref, b_ref, h_ref, rh_ref):
    z = jnp.dot(x_ref[...], w_ref[...], preferred_element_type=jnp.float32)
    z = z + b_ref[...]
    hz = z * jax.nn.sigmoid(z)
    h_ref[...] = hz
    rh_ref[...] = jnp.maximum(hz, 0.0)


_inproj = pl.pallas_call(
    _inproj_body,
    grid=(N // BLK,),
    in_specs=[
        pl.BlockSpec((BLK, H), lambda i: (i, 0)),
        pl.BlockSpec((H, H), lambda i: (0, 0)),
        pl.BlockSpec((1, H), lambda i: (0, 0)),
    ],
    out_specs=[pl.BlockSpec((BLK, H), lambda i: (i, 0))] * 2,
    out_shape=[jax.ShapeDtypeStruct((N, H), jnp.float32)] * 2,
)


def _layer_body(h_ref, p_ref, w_ref, b_ref, ga_ref, be_ref, out_ref, rh_ref):
    h0 = h_ref[...]
    p = p_ref[...]
    t = h0 + p[0] + p[1]
    z = jnp.dot(t, w_ref[...], preferred_element_type=jnp.float32)
    z = z + b_ref[...]
    mu = jnp.mean(z, axis=-1, keepdims=True)
    zc = z - mu
    var = jnp.mean(zc * zc, axis=-1, keepdims=True)
    zn = zc * lax.rsqrt(var + 1e-5) * ga_ref[...] + be_ref[...]
    zs = zn * jax.nn.sigmoid(zn)
    hn = zs + h0
    out_ref[...] = hn
    rh_ref[...] = jnp.maximum(hn, 0.0)


_layer = pl.pallas_call(
    _layer_body,
    grid=(N // BLK,),
    in_specs=[
        pl.BlockSpec((BLK, H), lambda i: (i, 0)),
        pl.BlockSpec((NC, BLK, H), lambda i: (0, i, 0)),
        pl.BlockSpec((H, H), lambda i: (0, 0)),
        pl.BlockSpec((1, H), lambda i: (0, 0)),
        pl.BlockSpec((1, H), lambda i: (0, 0)),
        pl.BlockSpec((1, H), lambda i: (0, 0)),
    ],
    out_specs=[pl.BlockSpec((BLK, H), lambda i: (i, 0))] * 2,
    out_shape=[jax.ShapeDtypeStruct((N, H), jnp.float32)] * 2,
)


def _pool_body(h_ref, batch_ref, out_ref):
    h = h_ref[...]
    bid = batch_ref[...]
    gids = lax.broadcasted_iota(jnp.int32, (N, G), 1)
    m = (bid == gids).astype(jnp.float32)
    sums = lax.dot_general(m, h, (((0,), (0,)), ((), ())),
                           preferred_element_type=jnp.float32)
    ones = jnp.ones((N, 1), jnp.float32)
    counts = lax.dot_general(m, ones, (((0,), (0,)), ((), ())),
                             preferred_element_type=jnp.float32)
    out_ref[...] = sums / jnp.maximum(counts, 1.0)


_pool = pl.pallas_call(
    _pool_body,
    out_shape=jax.ShapeDtypeStruct((G, H), jnp.float32),
)


def kernel(x, edge_index, batch, W_in, b_in, W, b, gamma, beta):
    src = edge_index[0]
    dst = edge_index[1]
    pad = EPAD - E
    src_p = jnp.concatenate([src, jnp.zeros((pad,), jnp.int32)])
    trash = N + jnp.arange(pad, dtype=jnp.int32) % (NPAD - N)
    dst_p = jnp.concatenate([dst, trash])
    srcv = src_p.reshape(NC, NS, NCHUNK // 2, 2 * CHUNK)
    dstv = dst_p.reshape(NC, NS, NCHUNK // 2, 2 * CHUNK)

    h, rh = _inproj(x, W_in, b_in.reshape(1, H))
    for l in range(L):
        parts = _sc_aggregate()(rh, srcv, dstv)
        h, rh = _layer(h, parts, W[l], b[l].reshape(1, H),
                       gamma[l].reshape(1, H), beta[l].reshape(1, H))
    return _pool(h, batch.reshape(N, 1))


# speedup vs baseline: 1.1837x; 1.1671x over previous
import functools

import jax
import jax.numpy as jnp
from jax import lax
from jax.experimental import pallas as pl
from jax.experimental.pallas import tpu as pltpu
from jax.experimental.pallas import tpu_sc as plsc

N = 10000
H = 128
E = 320000
G = 64
L = 3

NC = 2
NS = 16
NTILES = NC * NS
CHUNK = 64
NCHUNK = 160
EPAD = NTILES * NCHUNK * CHUNK
NPAD = 10240
ZROWS = NPAD // NS
RCHUNK = 128


def _sc_aggregate_body(rh_hbm, srcv_hbm, dstv_hbm, out_hbm,
                       srcv, dstv, rows, aggr, sem0, sem1):
    c = lax.axis_index("c")
    s = lax.axis_index("s")

    with jax.named_scope("sc_load_slabs"):
        pltpu.sync_copy(srcv_hbm.at[c, s], srcv)
        pltpu.sync_copy(dstv_hbm.at[c, s], dstv)

    z16 = jnp.zeros((16,), jnp.float32)

    def _zero_row(i, carry):
        for k in range(H // 16):
            rows[0, i, pl.ds(k * 16, 16)] = z16
        return carry

    with jax.named_scope("sc_zero"):
        lax.fori_loop(0, CHUNK, _zero_row, 0)

        def _zero_slab(k, carry):
            pltpu.sync_copy(rows.at[0],
                            aggr.at[pl.ds(s * ZROWS + k * CHUNK, CHUNK)])
            return carry

        lax.fori_loop(0, ZROWS // CHUNK, _zero_slab, 0)
        plsc.subcore_barrier()

    sems = (sem0, sem1)

    def _gather(row, half, b):
        pltpu.async_copy(
            rh_hbm.at[srcv.at[row, pl.ds(half * CHUNK, CHUNK)]],
            rows.at[b], sems[b])

    def _wait(b):
        pltpu.make_async_copy(
            rh_hbm.at[srcv.at[0, pl.ds(0, CHUNK)]],
            rows.at[b], sems[b]).wait()

    def _scatter(j, b):
        pltpu.sync_copy(rows.at[b], aggr.at[dstv.at[j]], add=True)

    with jax.named_scope("sc_mainloop"):
        _gather(0, 0, 0)
        _gather(0, 1, 1)

        def _round(g, carry):
            _wait(0)
            _scatter(2 * g, 0)

            @pl.when(g < NCHUNK // 2 - 1)
            def _():
                _gather(g + 1, 0, 0)

            _wait(1)
            _scatter(2 * g + 1, 1)

            @pl.when(g < NCHUNK // 2 - 1)
            def _():
                _gather(g + 1, 1, 1)

            return carry

        lax.fori_loop(0, NCHUNK // 2, _round, 0)
        plsc.subcore_barrier()

    with jax.named_scope("sc_writeback"):
        for k in range(ZROWS // RCHUNK):
            r0 = s * ZROWS + k * RCHUNK
            pltpu.sync_copy(aggr.at[pl.ds(r0, RCHUNK)],
                            out_hbm.at[c, pl.ds(r0, RCHUNK)])


@functools.cache
def _sc_aggregate():
    return pl.kernel(
        _sc_aggregate_body,
        out_type=jax.ShapeDtypeStruct((NC, NPAD, H), jnp.float32),
        mesh=plsc.VectorSubcoreMesh(core_axis_name="c", subcore_axis_name="s",
                                    num_cores=NC, num_subcores=NS),
        scratch_types=[
            pltpu.VMEM((NCHUNK // 2, 2 * CHUNK), jnp.int32),
            pltpu.VMEM((NCHUNK, CHUNK), jnp.int32),
            pltpu.VMEM((2, CHUNK, H), jnp.float32),
            pltpu.VMEM_SHARED((NPAD, H), jnp.float32),
            pltpu.SemaphoreType.DMA,
            pltpu.SemaphoreType.DMA,
        ],
    )


BLK = 2000


def _inproj_body(x_ref, w_ref, b_ref, h_ref, rh_ref):
    z = jnp.dot(x_ref[...], w_ref[...], preferred_element_type=jnp.float32)
    z = z + b_ref[...]
    hz = z * jax.nn.sigmoid(z)
    h_ref[...] = hz
    rh_ref[...] = jnp.maximum(hz, 0.0)


_inproj = pl.pallas_call(
    _inproj_body,
    grid=(N // BLK,),
    in_specs=[
        pl.BlockSpec((BLK, H), lambda i: (i, 0)),
        pl.BlockSpec((H, H), lambda i: (0, 0)),
        pl.BlockSpec((1, H), lambda i: (0, 0)),
    ],
    out_specs=[pl.BlockSpec((BLK, H), lambda i: (i, 0))] * 2,
    out_shape=[jax.ShapeDtypeStruct((N, H), jnp.float32)] * 2,
)


def _layer_body(h_ref, p_ref, w_ref, b_ref, ga_ref, be_ref, out_ref, rh_ref):
    h0 = h_ref[...]
    p = p_ref[...]
    t = h0 + p[0] + p[1]
    z = jnp.dot(t, w_ref[...], preferred_element_type=jnp.float32)
    z = z + b_ref[...]
    mu = jnp.mean(z, axis=-1, keepdims=True)
    zc = z - mu
    var = jnp.mean(zc * zc, axis=-1, keepdims=True)
    zn = zc * lax.rsqrt(var + 1e-5) * ga_ref[...] + be_ref[...]
    zs = zn * jax.nn.sigmoid(zn)
    hn = zs + h0
    out_ref[...] = hn
    rh_ref[...] = jnp.maximum(hn, 0.0)


_layer = pl.pallas_call(
    _layer_body,
    grid=(N // BLK,),
    in_specs=[
        pl.BlockSpec((BLK, H), lambda i: (i, 0)),
        pl.BlockSpec((NC, BLK, H), lambda i: (0, i, 0)),
        pl.BlockSpec((H, H), lambda i: (0, 0)),
        pl.BlockSpec((1, H), lambda i: (0, 0)),
        pl.BlockSpec((1, H), lambda i: (0, 0)),
        pl.BlockSpec((1, H), lambda i: (0, 0)),
    ],
    out_specs=[pl.BlockSpec((BLK, H), lambda i: (i, 0))] * 2,
    out_shape=[jax.ShapeDtypeStruct((N, H), jnp.float32)] * 2,
)


def _pool_body(h_ref, batch_ref, out_ref):
    h = h_ref[...]
    bid = batch_ref[...]
    gids = lax.broadcasted_iota(jnp.int32, (N, G), 1)
    m = (bid == gids).astype(jnp.float32)
    sums = lax.dot_general(m, h, (((0,), (0,)), ((), ())),
                           preferred_element_type=jnp.float32)
    ones = jnp.ones((N, 1), jnp.float32)
    counts = lax.dot_general(m, ones, (((0,), (0,)), ((), ())),
                             preferred_element_type=jnp.float32)
    out_ref[...] = sums / jnp.maximum(counts, 1.0)


_pool = pl.pallas_call(
    _pool_body,
    out_shape=jax.ShapeDtypeStruct((G, H), jnp.float32),
)


def kernel(x, edge_index, batch, W_in, b_in, W, b, gamma, beta):
    src = edge_index[0]
    dst = edge_index[1]
    pad = EPAD - E
    src_p = jnp.concatenate([src, jnp.zeros((pad,), jnp.int32)])
    trash = N + jnp.arange(pad, dtype=jnp.int32) % (NPAD - N)
    dst_p = jnp.concatenate([dst, trash])
    srcv = src_p.reshape(NC, NS, NCHUNK // 2, 2 * CHUNK)
    dstv = dst_p.reshape(NC, NS, NCHUNK, CHUNK)

    h, rh = _inproj(x, W_in, b_in.reshape(1, H))
    for l in range(L):
        parts = _sc_aggregate()(rh, srcv, dstv)
        h, rh = _layer(h, parts, W[l], b[l].reshape(1, H),
                       gamma[l].reshape(1, H), beta[l].reshape(1, H))
    return _pool(h, batch.reshape(N, 1))


# speedup vs baseline: 4.3191x; 3.6489x over previous
import functools

import jax
import jax.numpy as jnp
from jax import lax
from jax.experimental import pallas as pl
from jax.experimental.pallas import tpu as pltpu
from jax.experimental.pallas import tpu_sc as plsc

N = 10000
H = 128
E = 320000
G = 64
L = 3

NC = 2
NS = 16
NTILES = NC * NS
CHUNK = 64
NCHUNK = 160
EPAD = NTILES * NCHUNK * CHUNK
NPAD = 10240
ZROWS = NPAD // NS
RCHUNK = 128


def _sc_aggregate_body(rh_hbm, srcv_hbm, dstv_hbm, out_hbm,
                       srcv, dstv, rows, aggr, sem0, sem1):
    c = lax.axis_index("c")
    s = lax.axis_index("s")

    with jax.named_scope("sc_load_slabs"):
        pltpu.sync_copy(srcv_hbm.at[c, s], srcv)
        pltpu.sync_copy(dstv_hbm.at[c, s], dstv)

    z16 = jnp.zeros((16,), jnp.float32)

    def _zero_row(i, carry):
        for k in range(H // 16):
            rows[0, i, pl.ds(k * 16, 16)] = z16
        return carry

    with jax.named_scope("sc_zero"):
        lax.fori_loop(0, CHUNK, _zero_row, 0)

        def _zero_slab(k, carry):
            pltpu.sync_copy(rows.at[0],
                            aggr.at[pl.ds(s * ZROWS + k * CHUNK, CHUNK)])
            return carry

        lax.fori_loop(0, ZROWS // CHUNK, _zero_slab, 0)
        plsc.subcore_barrier()

    sems = (sem0, sem1)

    def _gather(row, half, b):
        pltpu.async_copy(
            rh_hbm.at[srcv.at[row, pl.ds(half * CHUNK, CHUNK)]],
            rows.at[b], sems[b])

    def _wait(b):
        pltpu.make_async_copy(
            rh_hbm.at[srcv.at[0, pl.ds(0, CHUNK)]],
            rows.at[b], sems[b]).wait()

    def _scatter(j, b):
        pltpu.sync_copy(rows.at[b], aggr.at[dstv.at[j]], add=True)

    with jax.named_scope("sc_mainloop"):
        _gather(0, 0, 0)
        _gather(0, 1, 1)

        def _round(g, carry):
            _wait(0)
            _scatter(2 * g, 0)

            @pl.when(g < NCHUNK // 2 - 1)
            def _():
                _gather(g + 1, 0, 0)

            _wait(1)
            _scatter(2 * g + 1, 1)

            @pl.when(g < NCHUNK // 2 - 1)
            def _():
                _gather(g + 1, 1, 1)

            return carry

        lax.fori_loop(0, NCHUNK // 2, _round, 0)
        plsc.subcore_barrier()

    with jax.named_scope("sc_writeback"):
        for k in range(ZROWS // RCHUNK):
            r0 = s * ZROWS + k * RCHUNK
            pltpu.sync_copy(aggr.at[pl.ds(r0, RCHUNK)],
                            out_hbm.at[c, pl.ds(r0, RCHUNK)])


@functools.cache
def _sc_aggregate():
    return pl.kernel(
        _sc_aggregate_body,
        out_type=jax.ShapeDtypeStruct((NC, NPAD, H), jnp.float32),
        mesh=plsc.VectorSubcoreMesh(core_axis_name="c", subcore_axis_name="s",
                                    num_cores=NC, num_subcores=NS),
        scratch_types=[
            pltpu.VMEM((NCHUNK // 2, 2 * CHUNK), jnp.int32),
            pltpu.VMEM((NCHUNK, CHUNK), jnp.int32),
            pltpu.VMEM((2, CHUNK, H), jnp.float32),
            pltpu.VMEM_SHARED((NPAD, H), jnp.float32),
            pltpu.SemaphoreType.DMA,
            pltpu.SemaphoreType.DMA,
        ],
    )


BLK = 2000


def _inproj_body(x_ref, w_ref, b_ref, h_ref, rh_ref):
    z = jnp.dot(x_ref[...], w_ref[...], preferred_element_type=jnp.float32)
    z = z + b_ref[...]
    hz = z * jax.nn.sigmoid(z)
    h_ref[...] = hz
    rh_ref[...] = jnp.maximum(hz, 0.0)


_inproj = pl.pallas_call(
    _inproj_body,
    grid=(N // BLK,),
    in_specs=[
        pl.BlockSpec((BLK, H), lambda i: (i, 0)),
        pl.BlockSpec((H, H), lambda i: (0, 0)),
        pl.BlockSpec((1, H), lambda i: (0, 0)),
    ],
    out_specs=[pl.BlockSpec((BLK, H), lambda i: (i, 0))] * 2,
    out_shape=[jax.ShapeDtypeStruct((N, H), jnp.float32)] * 2,
)


def _layer_body(h_ref, p_ref, w_ref, b_ref, ga_ref, be_ref, out_ref, rh_ref):
    h0 = h_ref[...]
    p = p_ref[...]
    t = h0 + p[0] + p[1]
    z = jnp.dot(t, w_ref[...], preferred_element_type=jnp.float32)
    z = z + b_ref[...]
    mu = jnp.mean(z, axis=-1, keepdims=True)
    zc = z - mu
    var = jnp.mean(zc * zc, axis=-1, keepdims=True)
    zn = zc * lax.rsqrt(var + 1e-5) * ga_ref[...] + be_ref[...]
    zs = zn * jax.nn.sigmoid(zn)
    hn = zs + h0
    out_ref[...] = hn
    rh_ref[...] = jnp.maximum(hn, 0.0)


_layer = pl.pallas_call(
    _layer_body,
    grid=(N // BLK,),
    in_specs=[
        pl.BlockSpec((BLK, H), lambda i: (i, 0)),
        pl.BlockSpec((NC, BLK, H), lambda i: (0, i, 0)),
        pl.BlockSpec((H, H), lambda i: (0, 0)),
        pl.BlockSpec((1, H), lambda i: (0, 0)),
        pl.BlockSpec((1, H), lambda i: (0, 0)),
        pl.BlockSpec((1, H), lambda i: (0, 0)),
    ],
    out_specs=[pl.BlockSpec((BLK, H), lambda i: (i, 0))] * 2,
    out_shape=[jax.ShapeDtypeStruct((N, H), jnp.float32)] * 2,
)


def _pool_body(h_ref, batch_ref, out_ref):
    h = h_ref[...]
    bid = batch_ref[...]
    gids = lax.broadcasted_iota(jnp.int32, (N, G), 1)
    m = (bid == gids).astype(jnp.float32)
    sums = lax.dot_general(m, h, (((0,), (0,)), ((), ())),
                           preferred_element_type=jnp.float32)
    ones = jnp.ones((N, 1), jnp.float32)
    counts = lax.dot_general(m, ones, (((0,), (0,)), ((), ())),
                             preferred_element_type=jnp.float32)
    out_ref[...] = sums / jnp.maximum(counts, 1.0)


_pool = pl.pallas_call(
    _pool_body,
    out_shape=jax.ShapeDtypeStruct((G, H), jnp.float32),
)


def kernel(x, edge_index, batch, W_in, b_in, W, b, gamma, beta):
    src = edge_index[0]
    dst = edge_index[1]
    pad = EPAD - E
    ar = jnp.arange(pad, dtype=jnp.int32)
    src_p = jnp.concatenate([src, ar % N])
    dst_p = jnp.concatenate([dst, N + ar % (NPAD - N)])
    srcv = src_p.reshape(NC, NS, NCHUNK // 2, 2 * CHUNK)
    dstv = dst_p.reshape(NC, NS, NCHUNK, CHUNK)

    h, rh = _inproj(x, W_in, b_in.reshape(1, H))
    for l in range(L):
        parts = _sc_aggregate()(rh, srcv, dstv)
        h, rh = _layer(h, parts, W[l], b[l].reshape(1, H),
                       gamma[l].reshape(1, H), beta[l].reshape(1, H))
    return _pool(h, batch.reshape(N, 1))


# speedup vs baseline: 4.3799x; 1.0141x over previous
import functools

import jax
import jax.numpy as jnp
from jax import lax
from jax.experimental import pallas as pl
from jax.experimental.pallas import tpu as pltpu
from jax.experimental.pallas import tpu_sc as plsc

N = 10000
H = 128
E = 320000
G = 64
L = 3

NC = 2
NS = 16
NTILES = NC * NS
CHUNK = 64
NCHUNK = 160
EPAD = NTILES * NCHUNK * CHUNK
NPAD = 10240
ZROWS = NPAD // NS
RCHUNK = 128


def _sc_aggregate_body(rh_hbm, srcv_hbm, dstv_hbm, out_hbm,
                       srcv, dstv, rows, aggr, sem0, sem1):
    c = lax.axis_index("c")
    s = lax.axis_index("s")

    with jax.named_scope("sc_load_slabs"):
        pltpu.sync_copy(srcv_hbm.at[c, s], srcv)
        pltpu.sync_copy(dstv_hbm.at[c, s], dstv)

    z16 = jnp.zeros((16,), jnp.float32)

    def _zero_row(i, carry):
        for k in range(H // 16):
            rows[0, i, pl.ds(k * 16, 16)] = z16
        return carry

    with jax.named_scope("sc_zero"):
        lax.fori_loop(0, CHUNK, _zero_row, 0)

        def _zero_slab(k, carry):
            pltpu.sync_copy(rows.at[0],
                            aggr.at[pl.ds(s * ZROWS + k * CHUNK, CHUNK)])
            return carry

        lax.fori_loop(0, ZROWS // CHUNK, _zero_slab, 0)
        plsc.subcore_barrier()

    sems = (sem0, sem1)

    def _gather(row, half, b):
        pltpu.async_copy(
            rh_hbm.at[srcv.at[row, pl.ds(half * CHUNK, CHUNK)]],
            rows.at[b], sems[b])

    def _wait(b):
        pltpu.make_async_copy(
            rh_hbm.at[srcv.at[0, pl.ds(0, CHUNK)]],
            rows.at[b], sems[b]).wait()

    def _scatter(j, b):
        pltpu.sync_copy(rows.at[b], aggr.at[dstv.at[j]], add=True)

    with jax.named_scope("sc_mainloop"):
        _gather(0, 0, 0)
        _gather(0, 1, 1)

        def _round(g, carry):
            _wait(0)
            _scatter(2 * g, 0)

            @pl.when(g < NCHUNK // 2 - 1)
            def _():
                _gather(g + 1, 0, 0)

            _wait(1)
            _scatter(2 * g + 1, 1)

            @pl.when(g < NCHUNK // 2 - 1)
            def _():
                _gather(g + 1, 1, 1)

            return carry

        lax.fori_loop(0, NCHUNK // 2, _round, 0)
        plsc.subcore_barrier()

    with jax.named_scope("sc_writeback"):
        for k in range(ZROWS // RCHUNK):
            r0 = s * ZROWS + k * RCHUNK
            pltpu.sync_copy(aggr.at[pl.ds(r0, RCHUNK)],
                            out_hbm.at[c, pl.ds(r0, RCHUNK)])


@functools.cache
def _sc_aggregate():
    return pl.kernel(
        _sc_aggregate_body,
        out_type=jax.ShapeDtypeStruct((NC, NPAD, H), jnp.float32),
        mesh=plsc.VectorSubcoreMesh(core_axis_name="c", subcore_axis_name="s",
                                    num_cores=NC, num_subcores=NS),
        scratch_types=[
            pltpu.VMEM((NCHUNK // 2, 2 * CHUNK), jnp.int32),
            pltpu.VMEM((NCHUNK, CHUNK), jnp.int32),
            pltpu.VMEM((2, CHUNK, H), jnp.float32),
            pltpu.VMEM_SHARED((NPAD, H), jnp.float32),
            pltpu.SemaphoreType.DMA,
            pltpu.SemaphoreType.DMA,
        ],
    )


BLK = 2000


def _inproj_body(x_ref, w_ref, b_ref, h_ref, rh_ref):
    z = jnp.dot(x_ref[...], w_ref[...], preferred_element_type=jnp.float32)
    z = z + b_ref[...]
    hz = z * jax.nn.sigmoid(z)
    h_ref[...] = hz
    rh_ref[...] = jnp.maximum(hz, 0.0)


_inproj = pl.pallas_call(
    _inproj_body,
    grid=(N // BLK,),
    in_specs=[
        pl.BlockSpec((BLK, H), lambda i: (i, 0)),
        pl.BlockSpec((H, H), lambda i: (0, 0)),
        pl.BlockSpec((1, H), lambda i: (0, 0)),
    ],
    out_specs=[pl.BlockSpec((BLK, H), lambda i: (i, 0))] * 2,
    out_shape=[jax.ShapeDtypeStruct((N, H), jnp.float32)] * 2,
)


def _layer_body(h_ref, p_ref, w_ref, b_ref, ga_ref, be_ref, out_ref, rh_ref):
    h0 = h_ref[...]
    p = p_ref[...]
    t = h0 + p[0] + p[1]
    z = jnp.dot(t, w_ref[...], preferred_element_type=jnp.float32)
    z = z + b_ref[...]
    mu = jnp.mean(z, axis=-1, keepdims=True)
    zc = z - mu
    var = jnp.mean(zc * zc, axis=-1, keepdims=True)
    zn = zc * lax.rsqrt(var + 1e-5) * ga_ref[...] + be_ref[...]
    zs = zn * jax.nn.sigmoid(zn)
    hn = zs + h0
    out_ref[...] = hn
    rh_ref[...] = jnp.maximum(hn, 0.0)


_layer = pl.pallas_call(
    _layer_body,
    grid=(N // BLK,),
    in_specs=[
        pl.BlockSpec((BLK, H), lambda i: (i, 0)),
        pl.BlockSpec((NC, BLK, H), lambda i: (0, i, 0)),
        pl.BlockSpec((H, H), lambda i: (0, 0)),
        pl.BlockSpec((1, H), lambda i: (0, 0)),
        pl.BlockSpec((1, H), lambda i: (0, 0)),
        pl.BlockSpec((1, H), lambda i: (0, 0)),
    ],
    out_specs=[pl.BlockSpec((BLK, H), lambda i: (i, 0))] * 2,
    out_shape=[jax.ShapeDtypeStruct((N, H), jnp.float32)] * 2,
)


def _layer_pool_body(h_ref, p_ref, w_ref, b_ref, ga_ref, be_ref, batch_ref,
                     out_ref, acc_s, acc_c):
    i = pl.program_id(0)
    h0 = h_ref[...]
    p = p_ref[...]
    t = h0 + p[0] + p[1]
    z = jnp.dot(t, w_ref[...], preferred_element_type=jnp.float32)
    z = z + b_ref[...]
    mu = jnp.mean(z, axis=-1, keepdims=True)
    zc = z - mu
    var = jnp.mean(zc * zc, axis=-1, keepdims=True)
    zn = zc * lax.rsqrt(var + 1e-5) * ga_ref[...] + be_ref[...]
    zs = zn * jax.nn.sigmoid(zn)
    hn = zs + h0
    gids = lax.broadcasted_iota(jnp.int32, (BLK, G), 1)
    m = (batch_ref[...] == gids).astype(jnp.float32)
    sums = lax.dot_general(m, hn, (((0,), (0,)), ((), ())),
                           preferred_element_type=jnp.float32)
    ones = jnp.ones((BLK, 1), jnp.float32)
    counts = lax.dot_general(m, ones, (((0,), (0,)), ((), ())),
                             preferred_element_type=jnp.float32)

    @pl.when(i == 0)
    def _():
        acc_s[...] = jnp.zeros_like(acc_s)
        acc_c[...] = jnp.zeros_like(acc_c)

    acc_s[...] += sums
    acc_c[...] += counts

    @pl.when(i == N // BLK - 1)
    def _():
        out_ref[...] = acc_s[...] / jnp.maximum(acc_c[...], 1.0)


_layer_pool = pl.pallas_call(
    _layer_pool_body,
    grid=(N // BLK,),
    in_specs=[
        pl.BlockSpec((BLK, H), lambda i: (i, 0)),
        pl.BlockSpec((NC, BLK, H), lambda i: (0, i, 0)),
        pl.BlockSpec((H, H), lambda i: (0, 0)),
        pl.BlockSpec((1, H), lambda i: (0, 0)),
        pl.BlockSpec((1, H), lambda i: (0, 0)),
        pl.BlockSpec((1, H), lambda i: (0, 0)),
        pl.BlockSpec((BLK, 1), lambda i: (i, 0)),
    ],
    out_specs=pl.BlockSpec((G, H), lambda i: (0, 0)),
    out_shape=jax.ShapeDtypeStruct((G, H), jnp.float32),
    scratch_shapes=[
        pltpu.VMEM((G, H), jnp.float32),
        pltpu.VMEM((G, 1), jnp.float32),
    ],
)


def kernel(x, edge_index, batch, W_in, b_in, W, b, gamma, beta):
    src = edge_index[0]
    dst = edge_index[1]
    pad = EPAD - E
    ar = jnp.arange(pad, dtype=jnp.int32)
    pad_cols = jnp.stack([ar % N, N + ar % (NPAD - N)])
    ei_p = jnp.concatenate([edge_index, pad_cols], axis=1)
    v = ei_p.reshape(2, NC, NS, NCHUNK // 2, 2 * CHUNK)
    srcv = v[0]
    dstv = v[1].reshape(NC, NS, NCHUNK, CHUNK)

    h, rh = _inproj(x, W_in, b_in.reshape(1, H))
    for l in range(L - 1):
        parts = _sc_aggregate()(rh, srcv, dstv)
        h, rh = _layer(h, parts, W[l], b[l].reshape(1, H),
                       gamma[l].reshape(1, H), beta[l].reshape(1, H))
    parts = _sc_aggregate()(rh, srcv, dstv)
    return _layer_pool(h, parts, W[L - 1], b[L - 1].reshape(1, H),
                       gamma[L - 1].reshape(1, H), beta[L - 1].reshape(1, H),
                       batch.reshape(N, 1))


# speedup vs baseline: 5.1903x; 1.1850x over previous
import functools

import jax
import jax.numpy as jnp
from jax import lax
from jax.experimental import pallas as pl
from jax.experimental.pallas import tpu as pltpu
from jax.experimental.pallas import tpu_sc as plsc

N = 10000
H = 128
E = 320000
G = 64
L = 3

NC = 2
NS = 16
NTILES = NC * NS
CHUNK = 128
NCHUNK = 80
EPAD = NTILES * NCHUNK * CHUNK
NPAD = 10240
ZROWS = NPAD // NS
RCHUNK = 128


def _sc_aggregate_body(rh_hbm, idx_hbm, out_hbm,
                       idxb, rows, aggr, gsem0, gsem1,
                       isem0, isem1, isem2, isem3):
    c = lax.axis_index("c")
    s = lax.axis_index("s")

    z16 = jnp.zeros((16,), jnp.float32)

    def _zero_row(i, carry):
        for k in range(H // 16):
            rows[0, i, pl.ds(k * 16, 16)] = z16
        return carry

    with jax.named_scope("sc_zero"):
        lax.fori_loop(0, CHUNK, _zero_row, 0)

        def _zero_slab(k, carry):
            pltpu.sync_copy(rows.at[0],
                            aggr.at[pl.ds(s * ZROWS + k * CHUNK, CHUNK)])
            return carry

        lax.fori_loop(0, ZROWS // CHUNK, _zero_slab, 0)
        plsc.subcore_barrier()

    gsems = (gsem0, gsem1)
    isems = (isem0, isem1, isem2, isem3)

    def _idx_fetch(j, ib):
        pltpu.async_copy(idx_hbm.at[c, s, j], idxb.at[ib], isems[ib])

    def _idx_wait(ib):
        pltpu.make_async_copy(idx_hbm.at[c, s, 0], idxb.at[ib],
                              isems[ib]).wait()

    def _gather(ib, b):
        pltpu.async_copy(rh_hbm.at[idxb.at[ib, 0]], rows.at[b], gsems[b])

    def _gather_wait(b):
        pltpu.make_async_copy(rh_hbm.at[idxb.at[0, 0]], rows.at[b],
                              gsems[b]).wait()

    def _scatter(ib, b):
        pltpu.sync_copy(rows.at[b], aggr.at[idxb.at[ib, 1]], add=True)

    with jax.named_scope("sc_mainloop"):
        for j0 in range(4):
            _idx_fetch(j0, j0)
        for j0 in range(2):
            _idx_wait(j0)
            _gather(j0, j0)

        def _round(g, carry):
            for slot in range(4):
                j = 4 * g + slot
                b = slot % 2
                _gather_wait(b)
                _scatter(slot, b)

                @pl.when(j + 4 < NCHUNK)
                def _():
                    _idx_fetch(j + 4, slot)

                @pl.when(j + 2 < NCHUNK)
                def _():
                    _idx_wait((slot + 2) % 4)
                    _gather((slot + 2) % 4, b)

            return carry

        lax.fori_loop(0, NCHUNK // 4, _round, 0)
        plsc.subcore_barrier()

    with jax.named_scope("sc_writeback"):
        for k in range(ZROWS // RCHUNK):
            r0 = s * ZROWS + k * RCHUNK
            pltpu.sync_copy(aggr.at[pl.ds(r0, RCHUNK)],
                            out_hbm.at[c, pl.ds(r0, RCHUNK)])


@functools.cache
def _sc_aggregate():
    return pl.kernel(
        _sc_aggregate_body,
        out_type=jax.ShapeDtypeStruct((NC, NPAD, H), jnp.float32),
        mesh=plsc.VectorSubcoreMesh(core_axis_name="c", subcore_axis_name="s",
                                    num_cores=NC, num_subcores=NS),
        scratch_types=[
            pltpu.VMEM((4, 2, CHUNK), jnp.int32),
            pltpu.VMEM((2, CHUNK, H), jnp.float32),
            pltpu.VMEM_SHARED((NPAD, H), jnp.float32),
            pltpu.SemaphoreType.DMA,
            pltpu.SemaphoreType.DMA,
            pltpu.SemaphoreType.DMA,
            pltpu.SemaphoreType.DMA,
            pltpu.SemaphoreType.DMA,
            pltpu.SemaphoreType.DMA,
        ],
    )


BLK = 2000


def _inproj_body(x_ref, w_ref, b_ref, h_ref, rh_ref):
    z = jnp.dot(x_ref[...], w_ref[...], preferred_element_type=jnp.float32)
    z = z + b_ref[...]
    hz = z * jax.nn.sigmoid(z)
    h_ref[...] = hz
    rh_ref[...] = jnp.maximum(hz, 0.0)


_inproj = pl.pallas_call(
    _inproj_body,
    grid=(N // BLK,),
    in_specs=[
        pl.BlockSpec((BLK, H), lambda i: (i, 0)),
        pl.BlockSpec((H, H), lambda i: (0, 0)),
        pl.BlockSpec((1, H), lambda i: (0, 0)),
    ],
    out_specs=[pl.BlockSpec((BLK, H), lambda i: (i, 0))] * 2,
    out_shape=[jax.ShapeDtypeStruct((N, H), jnp.float32)] * 2,
)


def _layer_body(h_ref, p_ref, w_ref, b_ref, ga_ref, be_ref, out_ref, rh_ref):
    h0 = h_ref[...]
    p = p_ref[...]
    t = h0 + p[0] + p[1]
    z = jnp.dot(t, w_ref[...], preferred_element_type=jnp.float32)
    z = z + b_ref[...]
    mu = jnp.mean(z, axis=-1, keepdims=True)
    zc = z - mu
    var = jnp.mean(zc * zc, axis=-1, keepdims=True)
    zn = zc * lax.rsqrt(var + 1e-5) * ga_ref[...] + be_ref[...]
    zs = zn * jax.nn.sigmoid(zn)
    hn = zs + h0
    out_ref[...] = hn
    rh_ref[...] = jnp.maximum(hn, 0.0)


_layer = pl.pallas_call(
    _layer_body,
    grid=(N // BLK,),
    in_specs=[
        pl.BlockSpec((BLK, H), lambda i: (i, 0)),
        pl.BlockSpec((NC, BLK, H), lambda i: (0, i, 0)),
        pl.BlockSpec((H, H), lambda i: (0, 0)),
        pl.BlockSpec((1, H), lambda i: (0, 0)),
        pl.BlockSpec((1, H), lambda i: (0, 0)),
        pl.BlockSpec((1, H), lambda i: (0, 0)),
    ],
    out_specs=[pl.BlockSpec((BLK, H), lambda i: (i, 0))] * 2,
    out_shape=[jax.ShapeDtypeStruct((N, H), jnp.float32)] * 2,
)


def _layer_pool_body(h_ref, p_ref, w_ref, b_ref, ga_ref, be_ref, batch_ref,
                     out_ref, acc_s, acc_c):
    i = pl.program_id(0)
    h0 = h_ref[...]
    p = p_ref[...]
    t = h0 + p[0] + p[1]
    z = jnp.dot(t, w_ref[...], preferred_element_type=jnp.float32)
    z = z + b_ref[...]
    mu = jnp.mean(z, axis=-1, keepdims=True)
    zc = z - mu
    var = jnp.mean(zc * zc, axis=-1, keepdims=True)
    zn = zc * lax.rsqrt(var + 1e-5) * ga_ref[...] + be_ref[...]
    zs = zn * jax.nn.sigmoid(zn)
    hn = zs + h0
    gids = lax.broadcasted_iota(jnp.int32, (BLK, G), 1)
    m = (batch_ref[...] == gids).astype(jnp.float32)
    sums = lax.dot_general(m, hn, (((0,), (0,)), ((), ())),
                           preferred_element_type=jnp.float32)
    ones = jnp.ones((BLK, 1), jnp.float32)
    counts = lax.dot_general(m, ones, (((0,), (0,)), ((), ())),
                             preferred_element_type=jnp.float32)

    @pl.when(i == 0)
    def _():
        acc_s[...] = jnp.zeros_like(acc_s)
        acc_c[...] = jnp.zeros_like(acc_c)

    acc_s[...] += sums
    acc_c[...] += counts

    @pl.when(i == N // BLK - 1)
    def _():
        out_ref[...] = acc_s[...] / jnp.maximum(acc_c[...], 1.0)


_layer_pool = pl.pallas_call(
    _layer_pool_body,
    grid=(N // BLK,),
    in_specs=[
        pl.BlockSpec((BLK, H), lambda i: (i, 0)),
        pl.BlockSpec((NC, BLK, H), lambda i: (0, i, 0)),
        pl.BlockSpec((H, H), lambda i: (0, 0)),
        pl.BlockSpec((1, H), lambda i: (0, 0)),
        pl.BlockSpec((1, H), lambda i: (0, 0)),
        pl.BlockSpec((1, H), lambda i: (0, 0)),
        pl.BlockSpec((BLK, 1), lambda i: (i, 0)),
    ],
    out_specs=pl.BlockSpec((G, H), lambda i: (0, 0)),
    out_shape=jax.ShapeDtypeStruct((G, H), jnp.float32),
    scratch_shapes=[
        pltpu.VMEM((G, H), jnp.float32),
        pltpu.VMEM((G, 1), jnp.float32),
    ],
)


def kernel(x, edge_index, batch, W_in, b_in, W, b, gamma, beta):
    pad = EPAD - E
    ar = jnp.arange(pad, dtype=jnp.int32)
    pad_cols = jnp.stack([ar % N, N + ar % (NPAD - N)])
    ei_p = jnp.concatenate([edge_index, pad_cols], axis=1)
    v = ei_p.reshape(2, NC, NS, NCHUNK, CHUNK)
    idx = jnp.stack([v[0], v[1]], axis=3)

    h, rh = _inproj(x, W_in, b_in.reshape(1, H))
    for l in range(L - 1):
        parts = _sc_aggregate()(rh, idx)
        h, rh = _layer(h, parts, W[l], b[l].reshape(1, H),
                       gamma[l].reshape(1, H), beta[l].reshape(1, H))
    parts = _sc_aggregate()(rh, idx)
    return _layer_pool(h, parts, W[L - 1], b[L - 1].reshape(1, H),
                       gamma[L - 1].reshape(1, H), beta[L - 1].reshape(1, H),
                       batch.reshape(N, 1))
